# Initial kernel scaffold; baseline (speedup 1.0000x reference)
import jax
import jax.numpy as jnp
from jax.experimental import pallas as pl


def kernel(x, W1, b1, W2, b2, batch):
    raise NotImplementedError("write your pallas kernel here")


# speedup vs baseline: 9.5463x; 9.5463x over previous
import functools

import jax
import jax.numpy as jnp
from jax.experimental import pallas as pl
from jax.experimental.pallas import tpu as pltpu

N_NODES = 100000
D = 512
H = 256
NSEG = 64
TILE = 2000
NTILES = N_NODES // TILE

NEG_INF = float("-inf")


def _body(x_ref, w1_ref, b1_ref, w2_ref, b2_ref, seg_ref, out_ref,
          acc_ref, m_ref, d_ref, cnt_ref):
    i = pl.program_id(0)

    @pl.when(i == 0)
    def _init():
        acc_ref[...] = jnp.zeros_like(acc_ref)
        m_ref[...] = jnp.full_like(m_ref, NEG_INF)
        d_ref[...] = jnp.zeros_like(d_ref)
        cnt_ref[...] = jnp.zeros_like(cnt_ref)

    xb = x_ref[...]
    xb16 = xb.astype(jnp.bfloat16)
    w1 = w1_ref[...].astype(jnp.bfloat16)
    h = jnp.dot(xb16, w1, preferred_element_type=jnp.float32)
    h = jnp.tanh(h + b1_ref[...])
    s = jnp.sum(h * w2_ref[...], axis=1, keepdims=True) + b2_ref[0, 0]

    seg = seg_ref[0]
    ids = jax.lax.broadcasted_iota(jnp.int32, (TILE, NSEG), 1)
    onehot = seg.reshape(TILE, 1) == ids

    tile_max = jnp.max(jnp.where(onehot, s, NEG_INF), axis=0, keepdims=True)
    m_old = m_ref[...]
    new_m = jnp.maximum(m_old, tile_max)
    m_safe = jnp.where(new_m == NEG_INF, 0.0, new_m)
    scale = jnp.where(m_old == NEG_INF, 0.0, jnp.exp(m_old - new_m))

    row_m = jnp.sum(jnp.where(onehot, m_safe, 0.0), axis=1, keepdims=True)
    ex = jnp.exp(s - row_m)
    p = jnp.where(onehot, ex, 0.0)

    d_ref[...] = d_ref[...] * scale + jnp.sum(p, axis=0, keepdims=True)
    cnt_ref[...] = cnt_ref[...] + jnp.sum(onehot.astype(jnp.float32), axis=0,
                                          keepdims=True)
    m_ref[...] = new_m

    contrib = jax.lax.dot_general(
        p.astype(jnp.bfloat16), xb16,
        dimension_numbers=(((0,), (0,)), ((), ())),
        preferred_element_type=jnp.float32)
    acc_ref[...] = acc_ref[...] * scale.reshape(NSEG, 1) + contrib

    @pl.when(i == NTILES - 1)
    def _fini():
        denom = d_ref[...].reshape(NSEG, 1) * cnt_ref[...].reshape(NSEG, 1)
        good = cnt_ref[...].reshape(NSEG, 1) > 0.0
        out_ref[...] = jnp.where(good, acc_ref[...] / jnp.where(good, denom, 1.0),
                                 0.0)


@jax.jit
def kernel(x, W1, b1, W2, b2, batch):
    seg = batch.astype(jnp.int32).reshape(NTILES, 1, TILE)
    b1r = b1.reshape(1, H).astype(jnp.float32)
    w2r = W2.reshape(1, H).astype(jnp.float32)
    b2r = b2.reshape(1, 1).astype(jnp.float32)
    grid = (NTILES,)
    out = pl.pallas_call(
        _body,
        grid=grid,
        in_specs=[
            pl.BlockSpec((TILE, D), lambda i: (i, 0)),
            pl.BlockSpec((D, H), lambda i: (0, 0)),
            pl.BlockSpec((1, H), lambda i: (0, 0)),
            pl.BlockSpec((1, H), lambda i: (0, 0)),
            pl.BlockSpec((1, 1), lambda i: (0, 0)),
            pl.BlockSpec((1, 1, TILE), lambda i: (i, 0, 0)),
        ],
        out_specs=pl.BlockSpec((NSEG, D), lambda i: (0, 0)),
        out_shape=jax.ShapeDtypeStruct((NSEG, D), jnp.float32),
        scratch_shapes=[
            pltpu.VMEM((NSEG, D), jnp.float32),
            pltpu.VMEM((1, NSEG), jnp.float32),
            pltpu.VMEM((1, NSEG), jnp.float32),
            pltpu.VMEM((1, NSEG), jnp.float32),
        ],
        compiler_params=pltpu.CompilerParams(
            dimension_semantics=("arbitrary",)),
    )(x, W1, b1r, w2r, b2r, seg)
    return out


# speedup vs baseline: 13.2197x; 1.3848x over previous
import jax
import jax.numpy as jnp
from jax.experimental import pallas as pl
from jax.experimental.pallas import tpu as pltpu

N_NODES = 100000
D = 512
H = 256
NSEG = 64
TILE = 2000
NTILES = N_NODES // TILE


def _body(x_ref, w1_ref, b1_ref, w2_ref, b2_ref, seg_ref, out_ref,
          acc_ref, d_ref, cnt_ref):
    i = pl.program_id(0)

    @pl.when(i == 0)
    def _init():
        acc_ref[...] = jnp.zeros_like(acc_ref)
        d_ref[...] = jnp.zeros_like(d_ref)
        cnt_ref[...] = jnp.zeros_like(cnt_ref)

    xb16 = x_ref[...].astype(jnp.bfloat16)
    w1 = w1_ref[...].astype(jnp.bfloat16)
    h = jnp.dot(xb16, w1, preferred_element_type=jnp.float32)
    h = jnp.tanh(h + b1_ref[...])
    w2 = w2_ref[...]
    b2 = b2_ref[0, 0]
    c = jnp.sum(jnp.abs(w2)) + jnp.abs(b2)
    s = jnp.sum(h * w2, axis=1, keepdims=True) + b2
    ex = jnp.exp(s - c)

    seg = seg_ref[0]
    ids = jax.lax.broadcasted_iota(jnp.int32, (TILE, NSEG), 1)
    mask = seg.reshape(TILE, 1) == ids

    q = jnp.where(mask, ex, 0.0)
    d_ref[...] = d_ref[...] + jnp.sum(q, axis=0, keepdims=True)
    cnt_ref[...] = cnt_ref[...] + jnp.sum(
        jnp.where(mask, 1.0, 0.0), axis=0, keepdims=True)

    contrib = jax.lax.dot_general(
        q.astype(jnp.bfloat16), xb16,
        dimension_numbers=(((0,), (0,)), ((), ())),
        preferred_element_type=jnp.float32)
    acc_ref[...] = acc_ref[...] + contrib

    @pl.when(i == NTILES - 1)
    def _fini():
        denom = d_ref[...].reshape(NSEG, 1) * cnt_ref[...].reshape(NSEG, 1)
        good = cnt_ref[...].reshape(NSEG, 1) > 0.0
        out_ref[...] = jnp.where(good, acc_ref[...] / jnp.where(good, denom, 1.0),
                                 0.0)


@jax.jit
def kernel(x, W1, b1, W2, b2, batch):
    seg = batch.astype(jnp.int32).reshape(NTILES, 1, TILE)
    b1r = b1.reshape(1, H).astype(jnp.float32)
    w2r = W2.reshape(1, H).astype(jnp.float32)
    b2r = b2.reshape(1, 1).astype(jnp.float32)
    out = pl.pallas_call(
        _body,
        grid=(NTILES,),
        in_specs=[
            pl.BlockSpec((TILE, D), lambda i: (i, 0)),
            pl.BlockSpec((D, H), lambda i: (0, 0)),
            pl.BlockSpec((1, H), lambda i: (0, 0)),
            pl.BlockSpec((1, H), lambda i: (0, 0)),
            pl.BlockSpec((1, 1), lambda i: (0, 0)),
            pl.BlockSpec((1, 1, TILE), lambda i: (i, 0, 0)),
        ],
        out_specs=pl.BlockSpec((NSEG, D), lambda i: (0, 0)),
        out_shape=jax.ShapeDtypeStruct((NSEG, D), jnp.float32),
        scratch_shapes=[
            pltpu.VMEM((NSEG, D), jnp.float32),
            pltpu.VMEM((1, NSEG), jnp.float32),
            pltpu.VMEM((1, NSEG), jnp.float32),
        ],
        compiler_params=pltpu.CompilerParams(
            dimension_semantics=("arbitrary",)),
    )(x, W1, b1r, w2r, b2r, seg)
    return out
